# baseline (speedup 1.0000x reference)
import functools

import jax
import jax.numpy as jnp
from jax import lax
from jax.experimental import pallas as pl
from jax.experimental.pallas import tpu as pltpu
from jax.experimental.pallas import tpu_sc as plsc

NN = 10000
EE = 320000
D = 128
NC, NS, L = 2, 16, 16
NW = NC * NS
NP = 10240
CH = 128
NCHK = 80
EPW = NCHK * CH
EPAD = NW * EPW
RPT = NP // NS
NR = NP // CH

_mesh = plsc.VectorSubcoreMesh(
    core_axis_name="c", subcore_axis_name="s", num_cores=NC, num_subcores=NS
)


@functools.partial(
    pl.kernel,
    out_type=jax.ShapeDtypeStruct((NW * NR, CH), jnp.float32),
    mesh=_mesh,
    scratch_types=[
        pltpu.VMEM((NCHK, CH), jnp.int32),
        pltpu.VMEM((NR, CH), jnp.float32),
    ],
    compiler_params=pltpu.CompilerParams(needs_layout_passes=False),
)
def _deg_kernel(dst_hbm, out_hbm, idx_v, hist_v):
    c = lax.axis_index("c")
    s = lax.axis_index("s")
    wid = s * NC + c

    pltpu.sync_copy(dst_hbm.at[pl.ds(wid * NCHK, NCHK)], idx_v)

    def fz(i, _):
        for k in range(CH // L):
            hist_v[i, pl.ds(k * L, L)] = jnp.zeros((L,), jnp.float32)
        return 0
    lax.fori_loop(0, NR, fz, 0)

    ones16 = jnp.ones((L,), jnp.float32)

    def fe(j, _):
        for k in range(CH // L):
            idx = idx_v[j, pl.ds(k * L, L)]
            row = lax.shift_right_logical(idx, 7)
            col = lax.bitwise_and(idx, 127)
            plsc.addupdate_scatter(hist_v, [row, col], ones16)
        return 0
    lax.fori_loop(0, NCHK, fe, 0)

    pltpu.sync_copy(hist_v, out_hbm.at[pl.ds(wid * NR, NR)])


NBUF = 3


QTR = NCHK // 4


@functools.partial(
    pl.kernel,
    out_type=jax.ShapeDtypeStruct((NC * NP, D), jnp.float32),
    mesh=_mesh,
    scratch_types=[
        pltpu.VMEM((QTR, CH), jnp.int32),
        pltpu.VMEM((QTR, CH), jnp.int32),
        pltpu.VMEM((NBUF, CH, D // 2), jnp.int32),
        pltpu.VMEM((CH, D), jnp.float32),
        pltpu.VMEM_SHARED((NP, D), jnp.float32),
        [pltpu.SemaphoreType.DMA] * NBUF,
        pltpu.SemaphoreType.DMA,
    ],
    compiler_params=pltpu.CompilerParams(needs_layout_passes=False,
                                         use_tc_tiling_on_sc=False),
)
def _mp_kernel(g_hbm, src_hbm, dst_hbm, out_hbm, si_v, di_v, rbf_v, rf_v,
               acc, gsem, ssem):
    c = lax.axis_index("c")
    s = lax.axis_index("s")
    wid = s * NC + c

    def fz(i, _):
        for k in range(D // L):
            rf_v[i, pl.ds(k * L, L)] = jnp.zeros((L,), jnp.float32)
        return 0
    lax.fori_loop(0, CH, fz, 0)

    def clear(t, _):
        pltpu.sync_copy(rf_v, acc.at[pl.ds(s * RPT + t * CH, CH)])
        return 0
    lax.fori_loop(0, RPT // CH, clear, 0)
    plsc.subcore_barrier()

    mask_hi = jnp.full((L,), -65536, jnp.int32)

    def _consume(t, b):
        pltpu.make_async_copy(g_hbm.at[si_v.at[t]], rbf_v.at[b],
                              gsem[b]).wait()

        def conv(r, _):
            for g in range(D // 32):
                u = rbf_v[b, r, pl.ds(g * L, L)]
                lo = plsc.bitcast(lax.shift_left(u, 16), jnp.float32)
                hi = plsc.bitcast(lax.bitwise_and(u, mask_hi), jnp.float32)
                rf_v[r, pl.ds(g * 32, L)] = lo
                rf_v[r, pl.ds(g * 32 + L, L)] = hi
            return 0
        lax.fori_loop(0, CH, conv, 0)

        pltpu.async_copy(rf_v, acc.at[di_v.at[t]], ssem, add=True)
        pltpu.make_async_copy(rf_v, acc.at[di_v.at[t]], ssem).wait()

        if not (isinstance(t, int) and t + NBUF >= QTR):
            @pl.when(t + NBUF < QTR)
            def _():
                pltpu.async_copy(g_hbm.at[si_v.at[t + NBUF]], rbf_v.at[b],
                                 gsem[b])

    for h in range(NCHK // QTR):
        base = wid * NCHK + h * QTR
        pltpu.sync_copy(src_hbm.at[pl.ds(base, QTR)], si_v)
        pltpu.sync_copy(dst_hbm.at[pl.ds(base, QTR)], di_v)
        for b in range(NBUF):
            pltpu.async_copy(g_hbm.at[si_v.at[b]], rbf_v.at[b], gsem[b])

        def step(tt, _):
            for b in range(NBUF):
                _consume(tt * NBUF + b, b)
            return 0
        lax.fori_loop(0, QTR // NBUF, step, 0)
        for t in range((QTR // NBUF) * NBUF, QTR):
            _consume(t, t % NBUF)
    plsc.subcore_barrier()

    def wout(t, _):
        pltpu.sync_copy(acc.at[pl.ds(s * RPT + t * CH, CH)], rf_v)
        pltpu.sync_copy(rf_v,
                        out_hbm.at[pl.ds(c * NP + s * RPT + t * CH, CH)])
        return 0
    lax.fori_loop(0, RPT // CH, wout, 0)


@functools.partial(
    pl.kernel,
    out_type=jax.ShapeDtypeStruct((NW * NCHK, CH), jnp.float32),
    mesh=_mesh,
    scratch_types=[
        pltpu.VMEM((NP * 2,), jnp.float32),
        pltpu.VMEM((NCHK, CH), jnp.int32),
        pltpu.VMEM((NCHK, CH), jnp.int32),
        pltpu.VMEM((NCHK, CH), jnp.float32),
    ],
    compiler_params=pltpu.CompilerParams(needs_layout_passes=False),
)
def _head_kernel(pq_hbm, src_hbm, dst_hbm, out_hbm, pq_v, si_v, di_v, o_v):
    c = lax.axis_index("c")
    s = lax.axis_index("s")
    wid = s * NC + c

    pltpu.sync_copy(pq_hbm, pq_v)
    pltpu.sync_copy(src_hbm.at[pl.ds(wid * NCHK, NCHK)], si_v)
    pltpu.sync_copy(dst_hbm.at[pl.ds(wid * NCHK, NCHK)], di_v)

    one16 = jnp.ones((L,), jnp.int32)

    def chunk(j, _):
        for k in range(CH // L):
            sl = pl.ds(k * L, L)
            sidx = si_v[j, sl]
            didx = di_v[j, sl]
            p = plsc.load_gather(pq_v, [sidx * 2])
            q = plsc.load_gather(pq_v, [didx * 2 + one16])
            o_v[j, sl] = p + q
        return 0
    lax.fori_loop(0, NCHK, chunk, 0)

    pltpu.sync_copy(o_v, out_hbm.at[pl.ds(wid * NCHK, NCHK)])


GB = 1024
NGB = NP // GB


def _pack_cols(g):
    a = jnp.concatenate([g[:, o : o + 16] for o in (0, 32, 64, 96)], axis=1)
    b = jnp.concatenate([g[:, o : o + 16] for o in (16, 48, 80, 112)], axis=1)
    au = lax.bitcast_convert_type(a.astype(jnp.bfloat16), jnp.uint16)
    bu = lax.bitcast_convert_type(b.astype(jnp.bfloat16), jnp.uint16)
    word = au.astype(jnp.uint32) | (bu.astype(jnp.uint32) << 16)
    return lax.bitcast_convert_type(word, jnp.int32)


def _tc_a_body(x_ref, w_ref, dg_ref, g_ref, dis_ref, gb_ref):
    deg = jnp.sum(dg_ref[:, :, :], axis=0) + 1.0
    dis_t = jnp.transpose(lax.rsqrt(deg))
    h = jnp.dot(x_ref[:, :], w_ref[:, :], preferred_element_type=jnp.float32)
    for j in range(GB // CH):
        dcol = dis_t[:, j : j + 1]
        lo, hi = j * CH, (j + 1) * CH
        g = dcol * h[lo:hi, :]
        g_ref[lo:hi, :] = g
        gb_ref[lo:hi, :] = _pack_cols(g)
        dis_ref[lo:hi, :] = dcol


_tc_a = pl.pallas_call(
    _tc_a_body,
    grid=(NGB,),
    in_specs=[
        pl.BlockSpec((GB, D), lambda i: (i, 0)),
        pl.BlockSpec((D, D), lambda i: (0, 0)),
        pl.BlockSpec((NW, GB // CH, CH), lambda i: (0, i, 0)),
    ],
    out_specs=[
        pl.BlockSpec((GB, D), lambda i: (i, 0)),
        pl.BlockSpec((GB, 1), lambda i: (i, 0)),
        pl.BlockSpec((GB, D // 2), lambda i: (i, 0)),
    ],
    out_shape=[
        jax.ShapeDtypeStruct((NP, D), jnp.float32),
        jax.ShapeDtypeStruct((NP, 1), jnp.float32),
        jax.ShapeDtypeStruct((NP, D // 2), jnp.int32),
    ],
)


def _tc_b_body(s_ref, g_ref, dis_ref, w_ref, b_ref, o_ref, ob_ref):
    dis = dis_ref[:, :]
    t = dis * (s_ref[0] + s_ref[1] + g_ref[:, :]) + b_ref[:, :]
    t = jnp.maximum(t, 0.0)
    h = jnp.dot(t, w_ref[:, :], preferred_element_type=jnp.float32)
    g = dis * h
    o_ref[:, :] = g
    ob_ref[:, :] = _pack_cols(g)


_tc_b = pl.pallas_call(
    _tc_b_body,
    grid=(NP // CH,),
    in_specs=[
        pl.BlockSpec((2, CH, D), lambda i: (0, i, 0)),
        pl.BlockSpec((CH, D), lambda i: (i, 0)),
        pl.BlockSpec((CH, 1), lambda i: (i, 0)),
        pl.BlockSpec((D, D), lambda i: (0, 0)),
        pl.BlockSpec((1, D), lambda i: (0, 0)),
    ],
    out_specs=[
        pl.BlockSpec((CH, D), lambda i: (i, 0)),
        pl.BlockSpec((CH, D // 2), lambda i: (i, 0)),
    ],
    out_shape=[
        jax.ShapeDtypeStruct((NP, D), jnp.float32),
        jax.ShapeDtypeStruct((NP, D // 2), jnp.int32),
    ],
)


def _tc_c_body(s_ref, g_ref, dis_ref, w_ref, b_ref, bfc_ref, o_ref):
    dis = dis_ref[:, :]
    t = dis * (s_ref[0] + s_ref[1] + g_ref[:, :]) + b_ref[:, :]
    t = jnp.maximum(t, 0.0)
    pq = jnp.dot(t, w_ref[:, :], preferred_element_type=jnp.float32)
    o_ref[:, :] = pq + 0.5 * bfc_ref[0, 0]


_tc_c = pl.pallas_call(
    _tc_c_body,
    grid=(NP // CH,),
    in_specs=[
        pl.BlockSpec((2, CH, D), lambda i: (0, i, 0)),
        pl.BlockSpec((CH, D), lambda i: (i, 0)),
        pl.BlockSpec((CH, 1), lambda i: (i, 0)),
        pl.BlockSpec((D, 2), lambda i: (0, 0)),
        pl.BlockSpec((1, D), lambda i: (0, 0)),
        pl.BlockSpec(memory_space=pltpu.SMEM),
    ],
    out_specs=pl.BlockSpec((CH, 2), lambda i: (i, 0)),
    out_shape=jax.ShapeDtypeStruct((NP, 2), jnp.float32),
)


def kernel(x, edge_index, W1, b1, W2, b2, Wfc, bfc):
    src = edge_index[0]
    dst = edge_index[1]
    pad = jnp.full((EPAD - EE,), NN, dtype=src.dtype)
    srcp = jnp.concatenate([src, pad]).reshape(NW * NCHK, CH)
    dstp = jnp.concatenate([dst, pad]).reshape(NW * NCHK, CH)
    xp = jnp.concatenate([x, jnp.zeros((NP - NN, D), x.dtype)], axis=0)

    dg = _deg_kernel(dstp).reshape(NW, NR, CH)
    g1, dis, g1b = _tc_a(xp, W1, dg)
    s1 = _mp_kernel(g1b, srcp, dstp).reshape(NC, NP, D)
    g2, g2b = _tc_b(s1, g1, dis, W2, b1.reshape(1, D))
    s2 = _mp_kernel(g2b, srcp, dstp).reshape(NC, NP, D)
    wpq = jnp.stack([Wfc[:D, 0], Wfc[D:, 0]], axis=1)
    pq = _tc_c(s2, g2, dis, wpq, b2.reshape(1, D), bfc.reshape(1, 1))
    pred = _head_kernel(pq.reshape(-1), srcp, dstp)
    return pred.reshape(-1)[:EE]


# speedup vs baseline: 1.0377x; 1.0377x over previous
import functools

import jax
import jax.numpy as jnp
from jax import lax
from jax.experimental import pallas as pl
from jax.experimental.pallas import tpu as pltpu
from jax.experimental.pallas import tpu_sc as plsc

NN = 10000
EE = 320000
D = 128
NC, NS, L = 2, 16, 16
NW = NC * NS
NP = 10240
CH = 128
NCHK = 80
EPW = NCHK * CH
EPAD = NW * EPW
RPT = NP // NS
NR = NP // CH

_mesh = plsc.VectorSubcoreMesh(
    core_axis_name="c", subcore_axis_name="s", num_cores=NC, num_subcores=NS
)


@functools.partial(
    pl.kernel,
    out_type=jax.ShapeDtypeStruct((NW * NR, CH), jnp.float32),
    mesh=_mesh,
    scratch_types=[
        pltpu.VMEM((NCHK, CH), jnp.int32),
        pltpu.VMEM((NR, CH), jnp.float32),
    ],
    compiler_params=pltpu.CompilerParams(needs_layout_passes=False),
)
def _deg_kernel(dst_hbm, out_hbm, idx_v, hist_v):
    c = lax.axis_index("c")
    s = lax.axis_index("s")
    wid = s * NC + c

    pltpu.sync_copy(dst_hbm.at[pl.ds(wid * NCHK, NCHK)], idx_v)

    def fz(i, _):
        for k in range(CH // L):
            hist_v[i, pl.ds(k * L, L)] = jnp.zeros((L,), jnp.float32)
        return 0
    lax.fori_loop(0, NR, fz, 0)

    ones16 = jnp.ones((L,), jnp.float32)

    def fe(j, _):
        for k in range(CH // L):
            idx = idx_v[j, pl.ds(k * L, L)]
            row = lax.shift_right_logical(idx, 7)
            col = lax.bitwise_and(idx, 127)
            plsc.addupdate_scatter(hist_v, [row, col], ones16)
        return 0
    lax.fori_loop(0, NCHK, fe, 0)

    pltpu.sync_copy(hist_v, out_hbm.at[pl.ds(wid * NR, NR)])


NBUF = 2


HALF = NCHK // 2


@functools.partial(
    pl.kernel,
    out_type=jax.ShapeDtypeStruct((NC * NP, D), jnp.float32),
    mesh=_mesh,
    scratch_types=[
        pltpu.VMEM((HALF, CH), jnp.int32),
        pltpu.VMEM((HALF, CH), jnp.int32),
        pltpu.VMEM((NBUF, CH, D // 2), jnp.int32),
        pltpu.VMEM((CH, D), jnp.float32),
        pltpu.VMEM_SHARED((NP, D), jnp.float32),
        [pltpu.SemaphoreType.DMA] * NBUF,
        pltpu.SemaphoreType.DMA,
    ],
    compiler_params=pltpu.CompilerParams(needs_layout_passes=False,
                                         use_tc_tiling_on_sc=False),
)
def _mp_kernel(g_hbm, src_hbm, dst_hbm, out_hbm, si_v, di_v, rbf_v, rf_v,
               acc, gsem, ssem):
    c = lax.axis_index("c")
    s = lax.axis_index("s")
    wid = s * NC + c

    def fz(i, _):
        for k in range(D // L):
            rf_v[i, pl.ds(k * L, L)] = jnp.zeros((L,), jnp.float32)
        return 0
    lax.fori_loop(0, CH, fz, 0)

    def clear(t, _):
        pltpu.sync_copy(rf_v, acc.at[pl.ds(s * RPT + t * CH, CH)])
        return 0
    lax.fori_loop(0, RPT // CH, clear, 0)
    plsc.subcore_barrier()

    mask_hi = jnp.full((L,), -65536, jnp.int32)

    for h in range(NCHK // HALF):
        base = wid * NCHK + h * HALF
        pltpu.sync_copy(src_hbm.at[pl.ds(base, HALF)], si_v)
        pltpu.sync_copy(dst_hbm.at[pl.ds(base, HALF)], di_v)
        for b in range(NBUF):
            pltpu.async_copy(g_hbm.at[si_v.at[b]], rbf_v.at[b], gsem[b])

        def step(tt, _):
            for b in range(NBUF):
                t = tt * NBUF + b
                pltpu.make_async_copy(g_hbm.at[si_v.at[t]], rbf_v.at[b],
                                      gsem[b]).wait()

                def conv(r, _):
                    for g in range(D // 32):
                        u = rbf_v[b, r, pl.ds(g * L, L)]
                        lo = plsc.bitcast(lax.shift_left(u, 16), jnp.float32)
                        hi = plsc.bitcast(lax.bitwise_and(u, mask_hi),
                                          jnp.float32)
                        rf_v[r, pl.ds(g * 32, L)] = lo
                        rf_v[r, pl.ds(g * 32 + L, L)] = hi
                    return 0
                lax.fori_loop(0, CH, conv, 0)

                pltpu.async_copy(rf_v, acc.at[di_v.at[t]], ssem, add=True)
                pltpu.make_async_copy(rf_v, acc.at[di_v.at[t]], ssem).wait()

                @pl.when(t + NBUF < HALF)
                def _():
                    pltpu.async_copy(g_hbm.at[si_v.at[t + NBUF]],
                                     rbf_v.at[b], gsem[b])
            return 0
        lax.fori_loop(0, HALF // NBUF, step, 0)
    plsc.subcore_barrier()

    def wout(t, _):
        pltpu.sync_copy(acc.at[pl.ds(s * RPT + t * CH, CH)], rf_v)
        pltpu.sync_copy(rf_v,
                        out_hbm.at[pl.ds(c * NP + s * RPT + t * CH, CH)])
        return 0
    lax.fori_loop(0, RPT // CH, wout, 0)


@functools.partial(
    pl.kernel,
    out_type=jax.ShapeDtypeStruct((NW * NCHK, CH), jnp.float32),
    mesh=_mesh,
    scratch_types=[
        pltpu.VMEM((NP * 2,), jnp.float32),
        pltpu.VMEM((NCHK, CH), jnp.int32),
        pltpu.VMEM((NCHK, CH), jnp.int32),
        pltpu.VMEM((NCHK, CH), jnp.float32),
    ],
    compiler_params=pltpu.CompilerParams(needs_layout_passes=False),
)
def _head_kernel(pq_hbm, src_hbm, dst_hbm, out_hbm, pq_v, si_v, di_v, o_v):
    c = lax.axis_index("c")
    s = lax.axis_index("s")
    wid = s * NC + c

    pltpu.sync_copy(pq_hbm, pq_v)
    pltpu.sync_copy(src_hbm.at[pl.ds(wid * NCHK, NCHK)], si_v)
    pltpu.sync_copy(dst_hbm.at[pl.ds(wid * NCHK, NCHK)], di_v)

    one16 = jnp.ones((L,), jnp.int32)

    def chunk(j, _):
        for k in range(CH // L):
            sl = pl.ds(k * L, L)
            sidx = si_v[j, sl]
            didx = di_v[j, sl]
            p = plsc.load_gather(pq_v, [sidx * 2])
            q = plsc.load_gather(pq_v, [didx * 2 + one16])
            o_v[j, sl] = p + q
        return 0
    lax.fori_loop(0, NCHK, chunk, 0)

    pltpu.sync_copy(o_v, out_hbm.at[pl.ds(wid * NCHK, NCHK)])


GB = 1024
NGB = NP // GB


def _pack_cols(g):
    a = jnp.concatenate([g[:, o : o + 16] for o in (0, 32, 64, 96)], axis=1)
    b = jnp.concatenate([g[:, o : o + 16] for o in (16, 48, 80, 112)], axis=1)
    au = lax.bitcast_convert_type(a.astype(jnp.bfloat16), jnp.uint16)
    bu = lax.bitcast_convert_type(b.astype(jnp.bfloat16), jnp.uint16)
    word = au.astype(jnp.uint32) | (bu.astype(jnp.uint32) << 16)
    return lax.bitcast_convert_type(word, jnp.int32)


def _tc_a_body(x_ref, w_ref, dg_ref, g_ref, dis_ref, gb_ref):
    deg = jnp.sum(dg_ref[:, :, :], axis=0) + 1.0
    dis_t = jnp.transpose(lax.rsqrt(deg))
    h = jnp.dot(x_ref[:, :], w_ref[:, :], preferred_element_type=jnp.float32)
    for j in range(GB // CH):
        dcol = dis_t[:, j : j + 1]
        lo, hi = j * CH, (j + 1) * CH
        g = dcol * h[lo:hi, :]
        g_ref[lo:hi, :] = g
        gb_ref[lo:hi, :] = _pack_cols(g)
        dis_ref[lo:hi, :] = dcol


_tc_a = pl.pallas_call(
    _tc_a_body,
    grid=(NGB,),
    in_specs=[
        pl.BlockSpec((GB, D), lambda i: (i, 0)),
        pl.BlockSpec((D, D), lambda i: (0, 0)),
        pl.BlockSpec((NW, GB // CH, CH), lambda i: (0, i, 0)),
    ],
    out_specs=[
        pl.BlockSpec((GB, D), lambda i: (i, 0)),
        pl.BlockSpec((GB, 1), lambda i: (i, 0)),
        pl.BlockSpec((GB, D // 2), lambda i: (i, 0)),
    ],
    out_shape=[
        jax.ShapeDtypeStruct((NP, D), jnp.float32),
        jax.ShapeDtypeStruct((NP, 1), jnp.float32),
        jax.ShapeDtypeStruct((NP, D // 2), jnp.int32),
    ],
)


def _tc_b_body(s_ref, g_ref, dis_ref, w_ref, b_ref, o_ref, ob_ref):
    dis = dis_ref[:, :]
    t = dis * (s_ref[0] + s_ref[1] + g_ref[:, :]) + b_ref[:, :]
    t = jnp.maximum(t, 0.0)
    h = jnp.dot(t, w_ref[:, :], preferred_element_type=jnp.float32)
    g = dis * h
    o_ref[:, :] = g
    ob_ref[:, :] = _pack_cols(g)


_tc_b = pl.pallas_call(
    _tc_b_body,
    grid=(NP // CH,),
    in_specs=[
        pl.BlockSpec((2, CH, D), lambda i: (0, i, 0)),
        pl.BlockSpec((CH, D), lambda i: (i, 0)),
        pl.BlockSpec((CH, 1), lambda i: (i, 0)),
        pl.BlockSpec((D, D), lambda i: (0, 0)),
        pl.BlockSpec((1, D), lambda i: (0, 0)),
    ],
    out_specs=[
        pl.BlockSpec((CH, D), lambda i: (i, 0)),
        pl.BlockSpec((CH, D // 2), lambda i: (i, 0)),
    ],
    out_shape=[
        jax.ShapeDtypeStruct((NP, D), jnp.float32),
        jax.ShapeDtypeStruct((NP, D // 2), jnp.int32),
    ],
)


def _tc_c_body(s_ref, g_ref, dis_ref, w_ref, b_ref, bfc_ref, o_ref):
    dis = dis_ref[:, :]
    t = dis * (s_ref[0] + s_ref[1] + g_ref[:, :]) + b_ref[:, :]
    t = jnp.maximum(t, 0.0)
    pq = jnp.dot(t, w_ref[:, :], preferred_element_type=jnp.float32)
    o_ref[:, :] = pq + 0.5 * bfc_ref[0, 0]


_tc_c = pl.pallas_call(
    _tc_c_body,
    grid=(NP // CH,),
    in_specs=[
        pl.BlockSpec((2, CH, D), lambda i: (0, i, 0)),
        pl.BlockSpec((CH, D), lambda i: (i, 0)),
        pl.BlockSpec((CH, 1), lambda i: (i, 0)),
        pl.BlockSpec((D, 2), lambda i: (0, 0)),
        pl.BlockSpec((1, D), lambda i: (0, 0)),
        pl.BlockSpec(memory_space=pltpu.SMEM),
    ],
    out_specs=pl.BlockSpec((CH, 2), lambda i: (i, 0)),
    out_shape=jax.ShapeDtypeStruct((NP, 2), jnp.float32),
)


def kernel(x, edge_index, W1, b1, W2, b2, Wfc, bfc):
    src = edge_index[0]
    dst = edge_index[1]
    pad = jnp.full((EPAD - EE,), NN, dtype=src.dtype)
    srcp = jnp.concatenate([src, pad]).reshape(NW * NCHK, CH)
    dstp = jnp.concatenate([dst, pad]).reshape(NW * NCHK, CH)
    xp = jnp.concatenate([x, jnp.zeros((NP - NN, D), x.dtype)], axis=0)

    dg = _deg_kernel(dstp).reshape(NW, NR, CH)
    g1, dis, g1b = _tc_a(xp, W1, dg)
    s1 = _mp_kernel(g1b, srcp, dstp).reshape(NC, NP, D)
    g2, g2b = _tc_b(s1, g1, dis, W2, b1.reshape(1, D))
    s2 = _mp_kernel(g2b, srcp, dstp).reshape(NC, NP, D)
    wpq = jnp.stack([Wfc[:D, 0], Wfc[D:, 0]], axis=1)
    pq = _tc_c(s2, g2, dis, wpq, b2.reshape(1, D), bfc.reshape(1, 1))
    pred = _head_kernel(pq.reshape(-1), srcp, dstp)
    return pred.reshape(-1)[:EE]
